# baseline (speedup 1.0000x reference)
import functools

import jax
import jax.numpy as jnp
from jax import lax
from jax.experimental import pallas as pl
from jax.experimental.pallas import tpu as pltpu
from jax.experimental.pallas import tpu_sc as plsc

_K = 6
_KS = 245
_NUM_CTX = sum(5 ** i for i in range(_K + 1))
_B, _L = 4096, 50
_NCOL = _L + 1
_R = _B * _NCOL
_KSP = 256

_NC, _NS = 2, 16
_NW = _NC * _NS
_BW = _B // _NW
_NCOLP = 56
_NG = _BW // 16
_IDXW = _BW * _NCOLP
_RP = _B * _NCOLP
_CH = 112
_NCHUNK = _IDXW // _CH

_P5 = [5 ** i for i in range(_K + 1)]
_OFFS = [(5 ** m - 1) // 4 for m in range(_K + 1)]


def _bias_body(e_ref, b_ref, o_ref):
    o_ref[:, : _KS] = e_ref[:] + b_ref[:]
    o_ref[:, _KS:] = jnp.zeros((o_ref.shape[0], _KSP - _KS), jnp.float32)


def _bias_table(emb, base_emb):
    rb = 1024
    grid = (_NUM_CTX + rb - 1) // rb
    return pl.pallas_call(
        _bias_body,
        grid=(grid,),
        in_specs=[
            pl.BlockSpec((rb, _KS), lambda i: (i, 0)),
            pl.BlockSpec((1, _KS), lambda i: (0, 0)),
        ],
        out_specs=pl.BlockSpec((rb, _KSP), lambda i: (i, 0)),
        out_shape=jax.ShapeDtypeStruct((_NUM_CTX, _KSP), jnp.float32),
    )(emb, base_emb.reshape(1, _KS))


def _compute_indices(x_v, idx_v):
    lanes = lax.iota(jnp.int32, 16)
    for g in range(_NG):
        rows = lanes + g * 16
        cols = [plsc.load_gather(x_v, [rows, jnp.full((16,), j, jnp.int32)])
                for j in range(_L)]
        base = (g * 16) * _NCOLP
        tgt0 = lanes * _NCOLP + base

        def put(p, vals):
            plsc.store_scatter(idx_v, [tgt0 + p], vals)

        v = jnp.zeros((16,), jnp.int32)
        put(0, v + _OFFS[0])
        for p in range(1, _K):
            v = v * 5 + cols[p - 1]
            put(p, v + _OFFS[p])
        w = v * 5 + cols[_K - 1]
        put(_K, w + _OFFS[_K])
        for p in range(_K + 1, _NCOL):
            w = (w - cols[p - 1 - _K] * _P5[_K - 1]) * 5 + cols[p - 1]
            put(p, w + _OFFS[_K])
        zeros = jnp.zeros((16,), jnp.int32)
        for p in range(_NCOL, _NCOLP):
            put(p, zeros)


def _sc_gather_body(tab_hbm, x_hbm, out_hbm, x_v, idx_v,
                    buf0, buf1, gsem0, gsem1, osem0, osem1):
    wid = lax.axis_index("s") * _NC + lax.axis_index("c")
    b0 = wid * _BW
    pltpu.sync_copy(x_hbm.at[pl.ds(b0, _BW)], x_v)
    _compute_indices(x_v, idx_v)

    bufs = (buf0, buf1)
    gsems = (gsem0, gsem1)
    osems = (osem0, osem1)
    base = wid * _IDXW

    def idx_at(ch):
        return idx_v.at[pl.ds(ch * _CH, _CH)]

    def out_at(ch):
        return out_hbm.at[pl.ds(base + ch * _CH, _CH)]

    pltpu.async_copy(tab_hbm.at[idx_at(0)], buf0, gsem0)
    pltpu.async_copy(tab_hbm.at[idx_at(1)], buf1, gsem1)

    def body(ch2, carry):
        for k in range(2):
            ch = ch2 * 2 + k
            pltpu.make_async_copy(tab_hbm.at[idx_at(ch)], bufs[k],
                                  gsems[k]).wait()
            pltpu.async_copy(bufs[k], out_at(ch), osems[k])
            pltpu.make_async_copy(bufs[k], out_at(ch), osems[k]).wait()

            @pl.when(ch + 2 < _NCHUNK)
            def _prefetch():
                pltpu.async_copy(tab_hbm.at[idx_at(ch + 2)], bufs[k],
                                 gsems[k])
        return carry

    lax.fori_loop(0, _NCHUNK // 2, body, 0)


_sc_gather = functools.partial(
    pl.kernel,
    mesh=plsc.VectorSubcoreMesh(core_axis_name="c", subcore_axis_name="s"),
    out_type=jax.ShapeDtypeStruct((_RP, _KSP), jnp.float32),
    scratch_types=[
        pltpu.VMEM((_BW, _L), jnp.int32),
        pltpu.VMEM((_IDXW,), jnp.int32),
        pltpu.VMEM((_CH, _KSP), jnp.float32),
        pltpu.VMEM((_CH, _KSP), jnp.float32),
        pltpu.SemaphoreType.DMA,
        pltpu.SemaphoreType.DMA,
        pltpu.SemaphoreType.DMA,
        pltpu.SemaphoreType.DMA,
    ],
    compiler_params=pltpu.CompilerParams(needs_layout_passes=False),
)(_sc_gather_body)


def kernel(x, emb, base_emb):
    x = x.astype(jnp.int32)
    emb_b = _bias_table(emb, base_emb)
    out = _sc_gather(emb_b, x)
    return out.reshape(_B, _NCOLP, _KSP)[:, : _NCOL, : _KS]


# speedup vs baseline: 2.3397x; 2.3397x over previous
import functools

import jax
import jax.numpy as jnp
from jax import lax
from jax.experimental import pallas as pl
from jax.experimental.pallas import tpu as pltpu
from jax.experimental.pallas import tpu_sc as plsc

_K = 6
_KS = 245
_NUM_CTX = sum(5 ** i for i in range(_K + 1))
_B, _L = 4096, 50
_NCOL = _L + 1
_R = _B * _NCOL
_KSP = 256

_NC, _NS = 2, 16
_NW = _NC * _NS
_BW = _B // _NW
_NCOLP = 56
_NG = _BW // 16
_IDXW = _BW * _NCOLP
_RP = _B * _NCOLP
_CH = 112
_NCHUNK = _IDXW // _CH

_P5 = [5 ** i for i in range(_K + 1)]
_OFFS = [(5 ** m - 1) // 4 for m in range(_K + 1)]


def _bias_body(e_ref, b_ref, o_ref):
    o_ref[:, : _KS] = e_ref[:] + b_ref[:]
    o_ref[:, _KS:] = jnp.zeros((o_ref.shape[0], _KSP - _KS), jnp.float32)


def _bias_table(emb, base_emb):
    rb = 1024
    grid = (_NUM_CTX + rb - 1) // rb
    return pl.pallas_call(
        _bias_body,
        grid=(grid,),
        in_specs=[
            pl.BlockSpec((rb, _KS), lambda i: (i, 0)),
            pl.BlockSpec((1, _KS), lambda i: (0, 0)),
        ],
        out_specs=pl.BlockSpec((rb, _KSP), lambda i: (i, 0)),
        out_shape=jax.ShapeDtypeStruct((_NUM_CTX, _KSP), jnp.float32),
    )(emb, base_emb.reshape(1, _KS))


def _compute_indices(x_v, idx_v):
    lanes = lax.iota(jnp.int32, 16)
    for g in range(_NG):
        rows = lanes + g * 16
        cols = [plsc.load_gather(x_v, [rows, jnp.full((16,), j, jnp.int32)])
                for j in range(_L)]
        base = (g * 16) * _NCOLP
        tgt0 = lanes * _NCOLP + base

        def put(p, vals):
            plsc.store_scatter(idx_v, [tgt0 + p], vals)

        v = jnp.zeros((16,), jnp.int32)
        put(0, v + _OFFS[0])
        for p in range(1, _K):
            v = v * 5 + cols[p - 1]
            put(p, v + _OFFS[p])
        w = v * 5 + cols[_K - 1]
        put(_K, w + _OFFS[_K])
        for p in range(_K + 1, _NCOL):
            w = (w - cols[p - 1 - _K] * _P5[_K - 1]) * 5 + cols[p - 1]
            put(p, w + _OFFS[_K])
        pad = w + _OFFS[_K]
        for p in range(_NCOL, _NCOLP):
            put(p, pad)


def _sc_gather_body(tab_hbm, x_hbm, out_hbm, x_v, idx_v,
                    buf0, buf1, gsem0, gsem1, osem0, osem1):
    wid = lax.axis_index("s") * _NC + lax.axis_index("c")
    b0 = wid * _BW
    pltpu.sync_copy(x_hbm.at[pl.ds(b0, _BW)], x_v)
    _compute_indices(x_v, idx_v)

    bufs = (buf0, buf1)
    gsems = (gsem0, gsem1)
    osems = (osem0, osem1)
    base = wid * _IDXW

    def idx_at(ch):
        return idx_v.at[pl.ds(ch * _CH, _CH)]

    def out_at(ch):
        return out_hbm.at[pl.ds(base + ch * _CH, _CH)]

    pltpu.async_copy(tab_hbm.at[idx_at(0)], buf0, gsem0)
    pltpu.async_copy(tab_hbm.at[idx_at(1)], buf1, gsem1)

    def body(ch2, carry):
        for k in range(2):
            ch = ch2 * 2 + k
            pltpu.make_async_copy(tab_hbm.at[idx_at(ch)], bufs[k],
                                  gsems[k]).wait()
            pltpu.async_copy(bufs[k], out_at(ch), osems[k])
            pltpu.make_async_copy(bufs[k], out_at(ch), osems[k]).wait()

            @pl.when(ch + 2 < _NCHUNK)
            def _prefetch():
                pltpu.async_copy(tab_hbm.at[idx_at(ch + 2)], bufs[k],
                                 gsems[k])
        return carry

    lax.fori_loop(0, _NCHUNK // 2, body, 0)


_sc_gather = functools.partial(
    pl.kernel,
    mesh=plsc.VectorSubcoreMesh(core_axis_name="c", subcore_axis_name="s"),
    out_type=jax.ShapeDtypeStruct((_RP, _KSP), jnp.float32),
    scratch_types=[
        pltpu.VMEM((_BW, _L), jnp.int32),
        pltpu.VMEM((_IDXW,), jnp.int32),
        pltpu.VMEM((_CH, _KSP), jnp.float32),
        pltpu.VMEM((_CH, _KSP), jnp.float32),
        pltpu.SemaphoreType.DMA,
        pltpu.SemaphoreType.DMA,
        pltpu.SemaphoreType.DMA,
        pltpu.SemaphoreType.DMA,
    ],
    compiler_params=pltpu.CompilerParams(needs_layout_passes=False),
)(_sc_gather_body)


def kernel(x, emb, base_emb):
    x = x.astype(jnp.int32)
    emb_b = _bias_table(emb, base_emb)
    out = _sc_gather(emb_b, x)
    return out.reshape(_B, _NCOLP, _KSP)[:, : _NCOL, : _KS]
